# baseline (speedup 1.0000x reference)
import jax
import jax.numpy as jnp
import numpy as np
from jax.experimental import pallas as pl
from jax.experimental.pallas import tpu as pltpu

_N = 20000
_NP = 20224
_H = _NP // 2
_G = 64
_T = 8


def _frnod_kernel(boxes_t_ref, scores_ref, gt_ref, gt_t_ref, loc_ref,
                  label_ref, max_ref):
    ax1 = boxes_t_ref[0:1, :]
    ay1 = boxes_t_ref[1:2, :]
    ax2 = boxes_t_ref[2:3, :]
    ay2 = boxes_t_ref[3:4, :]
    area_a = (ax2 - ax1) * (ay2 - ay1)

    zero = jnp.float32(0.0)
    a_iota = jax.lax.broadcasted_iota(jnp.int32, (_T, _H), 1)
    r_iota = jax.lax.broadcasted_iota(jnp.int32, (_T, _H), 0)

    max8 = jnp.full((_T, _H), -1.0, dtype=jnp.float32)
    tix8 = jnp.zeros((_T, _H), dtype=jnp.int32)
    col_args = []
    for k in range(_G // _T):
        g0 = k * _T
        gx1 = gt_ref[g0:g0 + _T, 0:1]
        gy1 = gt_ref[g0:g0 + _T, 1:2]
        gx2 = gt_ref[g0:g0 + _T, 2:3]
        gy2 = gt_ref[g0:g0 + _T, 3:4]
        iw = jnp.maximum(jnp.minimum(ax2, gx2) - jnp.maximum(ax1, gx1), zero)
        ih = jnp.maximum(jnp.minimum(ay2, gy2) - jnp.maximum(ay1, gy1), zero)
        area_i = iw * ih
        area_b = (gx2 - gx1) * (gy2 - gy1)
        iou = area_i / (area_a + area_b - area_i)
        better = iou > max8
        max8 = jnp.where(better, iou, max8)
        tix8 = jnp.where(better, k, tix8)
        cm = jnp.max(iou, axis=1, keepdims=True)
        ca = jnp.min(jnp.where(iou == cm, a_iota, _N), axis=1, keepdims=True)
        col_args.append(ca)

    max_iou = jnp.max(max8, axis=0, keepdims=True)
    cand = jnp.where(max8 == max_iou, tix8 * _T + r_iota, _G)
    argmax = jnp.min(cand, axis=0, keepdims=True)

    gt_arg = jnp.concatenate(col_args, axis=0)

    gsel8 = jnp.full((_T, _H), -1, dtype=jnp.int32)
    for k in range(_G // _T):
        g0 = k * _T
        match = a_iota == gt_arg[g0:g0 + _T, :]
        gsel8 = jnp.maximum(gsel8, jnp.where(match, r_iota + g0, -1))
    g_sel = jnp.max(gsel8, axis=0, keepdims=True)
    final_arg = jnp.where(g_sel >= 0, g_sel, argmax)

    lab = jnp.where(max_iou < 0.3, 0, -1)
    lab = jnp.where(max_iou >= 0.7, 1, lab)
    lab = jnp.where(g_sel >= 0, 1, lab)
    label_ref[...] = lab
    max_ref[...] = max_iou

    g_iota = jax.lax.broadcasted_iota(jnp.int32, (_G, _H), 0)
    onehot = (g_iota == final_arg).astype(jnp.float32)
    assigned = jax.lax.dot_general(
        gt_t_ref[...], onehot,
        dimension_numbers=(((1,), (0,)), ((), ())),
        precision=jax.lax.Precision.HIGHEST,
        preferred_element_type=jnp.float32,
    )
    bx1 = assigned[0:1, :]
    by1 = assigned[1:2, :]
    bx2 = assigned[2:3, :]
    by2 = assigned[3:4, :]

    width = ax2 - ax1
    height = ay2 - ay1
    ctr_x = ax1 + 0.5 * width
    ctr_y = ay1 + 0.5 * height
    base_w = bx2 - bx1
    base_h = by2 - by1
    base_cx = bx1 + 0.5 * base_w
    base_cy = by1 + 0.5 * base_h
    eps = jnp.float32(np.finfo(np.float32).eps)
    width = jnp.maximum(width, eps)
    height = jnp.maximum(height, eps)
    dx = (base_cx - ctr_x) / width
    dy = (base_cy - ctr_y) / height
    dw = jnp.log(base_w / width)
    dh = jnp.log(base_h / height)
    loc = jnp.concatenate([dx, dy, dw, dh], axis=0)
    loc_ref[...] = loc * scores_ref[...]


def kernel(boxes, scores, gt_boxes):
    boxes_t = jnp.pad(boxes.T, ((0, 0), (0, _NP - _N)))
    scores2 = jnp.pad(scores.reshape(1, _N), ((0, 0), (0, _NP - _N)))
    loc_t, label, max_ious = pl.pallas_call(
        _frnod_kernel,
        grid=(2,),
        in_specs=[
            pl.BlockSpec((4, _H), lambda i: (0, i)),
            pl.BlockSpec((1, _H), lambda i: (0, i)),
            pl.BlockSpec((_G, 4), lambda i: (0, 0)),
            pl.BlockSpec((4, _G), lambda i: (0, 0)),
        ],
        out_specs=[
            pl.BlockSpec((4, _H), lambda i: (0, i)),
            pl.BlockSpec((1, _H), lambda i: (0, i)),
            pl.BlockSpec((1, _H), lambda i: (0, i)),
        ],
        out_shape=[
            jax.ShapeDtypeStruct((4, _NP), jnp.float32),
            jax.ShapeDtypeStruct((1, _NP), jnp.int32),
            jax.ShapeDtypeStruct((1, _NP), jnp.float32),
        ],
        compiler_params=pltpu.CompilerParams(
            dimension_semantics=("parallel",),
        ),
    )(boxes_t, scores2, gt_boxes, gt_boxes.T)
    return loc_t[:, :_N].T, label[0, :_N], max_ious[0, :_N]


# speedup vs baseline: 1.1013x; 1.1013x over previous
import jax
import jax.numpy as jnp
import numpy as np
from jax.experimental import pallas as pl

_N = 20000
_G = 64
_T = 8


def _frnod_kernel(boxes_t_ref, scores_ref, gt_ref, gt_t_ref, loc_ref,
                  label_ref, max_ref):
    ax1 = boxes_t_ref[0:1, :]
    ay1 = boxes_t_ref[1:2, :]
    ax2 = boxes_t_ref[2:3, :]
    ay2 = boxes_t_ref[3:4, :]
    area_a = (ax2 - ax1) * (ay2 - ay1)

    zero = jnp.float32(0.0)
    a_iota = jax.lax.broadcasted_iota(jnp.int32, (_T, _N), 1)
    r_iota = jax.lax.broadcasted_iota(jnp.int32, (_T, _N), 0)

    max8 = jnp.full((_T, _N), -1.0, dtype=jnp.float32)
    tix8 = jnp.zeros((_T, _N), dtype=jnp.int32)
    col_args = []
    for k in range(_G // _T):
        g0 = k * _T
        gx1 = gt_ref[g0:g0 + _T, 0:1]
        gy1 = gt_ref[g0:g0 + _T, 1:2]
        gx2 = gt_ref[g0:g0 + _T, 2:3]
        gy2 = gt_ref[g0:g0 + _T, 3:4]
        iw = jnp.maximum(jnp.minimum(ax2, gx2) - jnp.maximum(ax1, gx1), zero)
        ih = jnp.maximum(jnp.minimum(ay2, gy2) - jnp.maximum(ay1, gy1), zero)
        area_i = iw * ih
        area_b = (gx2 - gx1) * (gy2 - gy1)
        iou = area_i / (area_a + area_b - area_i)
        better = iou > max8
        max8 = jnp.where(better, iou, max8)
        tix8 = jnp.where(better, k, tix8)
        cm = jnp.max(iou, axis=1, keepdims=True)
        ca = jnp.min(jnp.where(iou == cm, a_iota, _N), axis=1, keepdims=True)
        col_args.append(ca)

    max_iou = jnp.max(max8, axis=0, keepdims=True)
    cand = jnp.where(max8 == max_iou, tix8 * _T + r_iota, _G)
    argmax = jnp.min(cand, axis=0, keepdims=True)

    gt_arg = jnp.concatenate(col_args, axis=0)

    gsel8 = jnp.full((_T, _N), -1, dtype=jnp.int32)
    for k in range(_G // _T):
        g0 = k * _T
        match = a_iota == gt_arg[g0:g0 + _T, :]
        gsel8 = jnp.maximum(gsel8, jnp.where(match, r_iota + g0, -1))
    g_sel = jnp.max(gsel8, axis=0, keepdims=True)
    final_arg = jnp.where(g_sel >= 0, g_sel, argmax)

    lab = jnp.where(max_iou < 0.3, 0, -1)
    lab = jnp.where(max_iou >= 0.7, 1, lab)
    lab = jnp.where(g_sel >= 0, 1, lab)
    label_ref[...] = lab
    max_ref[...] = max_iou

    g_iota = jax.lax.broadcasted_iota(jnp.int32, (_G, _N), 0)
    onehot = (g_iota == final_arg).astype(jnp.float32)
    assigned = jax.lax.dot_general(
        gt_t_ref[...], onehot,
        dimension_numbers=(((1,), (0,)), ((), ())),
        precision=jax.lax.Precision.HIGHEST,
        preferred_element_type=jnp.float32,
    )
    bx1 = assigned[0:1, :]
    by1 = assigned[1:2, :]
    bx2 = assigned[2:3, :]
    by2 = assigned[3:4, :]

    width = ax2 - ax1
    height = ay2 - ay1
    ctr_x = ax1 + 0.5 * width
    ctr_y = ay1 + 0.5 * height
    base_w = bx2 - bx1
    base_h = by2 - by1
    base_cx = bx1 + 0.5 * base_w
    base_cy = by1 + 0.5 * base_h
    eps = jnp.float32(np.finfo(np.float32).eps)
    width = jnp.maximum(width, eps)
    height = jnp.maximum(height, eps)
    dx = (base_cx - ctr_x) / width
    dy = (base_cy - ctr_y) / height
    dw = jnp.log(base_w / width)
    dh = jnp.log(base_h / height)
    loc = jnp.concatenate([dx, dy, dw, dh], axis=0)
    loc_ref[...] = loc * scores_ref[...]


def kernel(boxes, scores, gt_boxes):
    boxes_t = boxes.T
    scores2 = scores.reshape(1, _N)
    loc_t, label, max_ious = pl.pallas_call(
        _frnod_kernel,
        out_shape=[
            jax.ShapeDtypeStruct((4, _N), jnp.float32),
            jax.ShapeDtypeStruct((1, _N), jnp.int32),
            jax.ShapeDtypeStruct((1, _N), jnp.float32),
        ],
    )(boxes_t, scores2, gt_boxes, gt_boxes.T)
    return loc_t.T, label.reshape(_N), max_ious.reshape(_N)


# speedup vs baseline: 1.1242x; 1.0208x over previous
import jax
import jax.numpy as jnp
import numpy as np
from jax.experimental import pallas as pl

_N = 20000
_G = 64
_T = 8


def _frnod_kernel(boxes_t_ref, scores_ref, gt_ref, gt_t_ref, loc_ref,
                  label_ref, max_ref):
    ax1 = boxes_t_ref[0:1, :]
    ay1 = boxes_t_ref[1:2, :]
    ax2 = boxes_t_ref[2:3, :]
    ay2 = boxes_t_ref[3:4, :]
    area_a = (ax2 - ax1) * (ay2 - ay1)

    zero = jnp.float32(0.0)
    a_iota = jax.lax.broadcasted_iota(jnp.int32, (_T, _N), 1)
    r_iota = jax.lax.broadcasted_iota(jnp.int32, (_T, _N), 0)

    max8 = jnp.full((_T, _N), -1.0, dtype=jnp.float32)
    tix8 = jnp.zeros((_T, _N), dtype=jnp.int32)
    col_args = []
    for k in range(_G // _T):
        g0 = k * _T
        gx1 = gt_ref[g0:g0 + _T, 0:1]
        gy1 = gt_ref[g0:g0 + _T, 1:2]
        gx2 = gt_ref[g0:g0 + _T, 2:3]
        gy2 = gt_ref[g0:g0 + _T, 3:4]
        iw = jnp.maximum(jnp.minimum(ax2, gx2) - jnp.maximum(ax1, gx1), zero)
        ih = jnp.maximum(jnp.minimum(ay2, gy2) - jnp.maximum(ay1, gy1), zero)
        area_i = iw * ih
        area_b = (gx2 - gx1) * (gy2 - gy1)
        iou = area_i / (area_a + area_b - area_i)
        better = iou > max8
        max8 = jnp.where(better, iou, max8)
        tix8 = jnp.where(better, k, tix8)
        cm = jnp.max(iou, axis=1, keepdims=True)
        ca = jnp.min(jnp.where(iou == cm, a_iota, _N), axis=1, keepdims=True)
        col_args.append(ca)

    max_iou = jnp.max(max8, axis=0, keepdims=True)
    cand = jnp.where(max8 == max_iou, tix8 * _T + r_iota, _G)
    argmax = jnp.min(cand, axis=0, keepdims=True)

    gt_arg = jnp.concatenate(col_args, axis=0)

    gsel8 = jnp.full((_T, _N), -1, dtype=jnp.int32)
    for k in range(_G // _T):
        g0 = k * _T
        match = a_iota == gt_arg[g0:g0 + _T, :]
        gsel8 = jnp.where(match, r_iota + g0, gsel8)
    g_sel = jnp.max(gsel8, axis=0, keepdims=True)
    final_arg = jnp.where(g_sel >= 0, g_sel, argmax)

    lab = jnp.where(max_iou < 0.3, 0, -1)
    lab = jnp.where(max_iou >= 0.7, 1, lab)
    lab = jnp.where(g_sel >= 0, 1, lab)
    label_ref[...] = lab
    max_ref[...] = max_iou

    g_iota = jax.lax.broadcasted_iota(jnp.int32, (_G, _N), 0)
    onehot = (g_iota == final_arg).astype(jnp.float32)
    assigned = jax.lax.dot_general(
        gt_t_ref[...], onehot,
        dimension_numbers=(((1,), (0,)), ((), ())),
        precision=jax.lax.Precision.HIGHEST,
        preferred_element_type=jnp.float32,
    )
    bx1 = assigned[0:1, :]
    by1 = assigned[1:2, :]
    bx2 = assigned[2:3, :]
    by2 = assigned[3:4, :]

    width = ax2 - ax1
    height = ay2 - ay1
    ctr_x = ax1 + 0.5 * width
    ctr_y = ay1 + 0.5 * height
    base_w = bx2 - bx1
    base_h = by2 - by1
    base_cx = bx1 + 0.5 * base_w
    base_cy = by1 + 0.5 * base_h
    eps = jnp.float32(np.finfo(np.float32).eps)
    width = jnp.maximum(width, eps)
    height = jnp.maximum(height, eps)
    dx = (base_cx - ctr_x) / width
    dy = (base_cy - ctr_y) / height
    dw = jnp.log(base_w / width)
    dh = jnp.log(base_h / height)
    loc = jnp.concatenate([dx, dy, dw, dh], axis=0)
    loc_ref[...] = loc * scores_ref[...]


def kernel(boxes, scores, gt_boxes):
    boxes_t = boxes.T
    scores2 = scores.reshape(1, _N)
    loc_t, label, max_ious = pl.pallas_call(
        _frnod_kernel,
        out_shape=[
            jax.ShapeDtypeStruct((4, _N), jnp.float32),
            jax.ShapeDtypeStruct((1, _N), jnp.int32),
            jax.ShapeDtypeStruct((1, _N), jnp.float32),
        ],
    )(boxes_t, scores2, gt_boxes, gt_boxes.T)
    return loc_t.T, label.reshape(_N), max_ious.reshape(_N)


# speedup vs baseline: 1.1336x; 1.0084x over previous
import jax
import jax.numpy as jnp
import numpy as np
from jax.experimental import pallas as pl

_N = 20000
_G = 64
_T = 8


def _frnod_kernel(boxes_t_ref, scores_ref, gt_ref, gt_t_ref, loc_ref,
                  label_ref, max_ref):
    ax1 = boxes_t_ref[0:1, :]
    ay1 = boxes_t_ref[1:2, :]
    ax2 = boxes_t_ref[2:3, :]
    ay2 = boxes_t_ref[3:4, :]
    area_a = (ax2 - ax1) * (ay2 - ay1)

    zero = jnp.float32(0.0)
    a_iota = jax.lax.broadcasted_iota(jnp.int32, (_T, _N), 1)
    r_iota = jax.lax.broadcasted_iota(jnp.int32, (_T, _N), 0)

    max8 = jnp.full((_T, _N), -1.0, dtype=jnp.float32)
    tix8 = jnp.zeros((_T, _N), dtype=jnp.int32)
    col_args = []
    for k in range(_G // _T):
        g0 = k * _T
        gx1 = gt_ref[g0:g0 + _T, 0:1]
        gy1 = gt_ref[g0:g0 + _T, 1:2]
        gx2 = gt_ref[g0:g0 + _T, 2:3]
        gy2 = gt_ref[g0:g0 + _T, 3:4]
        iw = jnp.maximum(jnp.minimum(ax2, gx2) - jnp.maximum(ax1, gx1), zero)
        ih = jnp.maximum(jnp.minimum(ay2, gy2) - jnp.maximum(ay1, gy1), zero)
        area_i = iw * ih
        area_b = (gx2 - gx1) * (gy2 - gy1)
        iou = area_i / (area_a + area_b - area_i)
        better = iou > max8
        max8 = jnp.where(better, iou, max8)
        tix8 = jnp.where(better, k, tix8)
        cm = jnp.max(iou, axis=1, keepdims=True)
        ca = jnp.min(jnp.where(iou == cm, a_iota, _N), axis=1, keepdims=True)
        col_args.append(ca)

    max_iou = jnp.max(max8, axis=0, keepdims=True)
    cand = jnp.where(max8 == max_iou, tix8 * _T + r_iota, _G)
    argmax = jnp.min(cand, axis=0, keepdims=True)

    gt_arg = jnp.concatenate(col_args, axis=0)

    gsel8 = jnp.full((_T, _N), -1, dtype=jnp.int32)
    for k in range(_G // _T):
        g0 = k * _T
        match = a_iota == gt_arg[g0:g0 + _T, :]
        gsel8 = jnp.where(match, r_iota + g0, gsel8)
    g_sel = jnp.max(gsel8, axis=0, keepdims=True)
    final_arg = jnp.where(g_sel >= 0, g_sel, argmax)

    lab = jnp.where(max_iou < 0.3, 0, -1)
    lab = jnp.where(max_iou >= 0.7, 1, lab)
    lab = jnp.where(g_sel >= 0, 1, lab)
    label_ref[...] = lab
    max_ref[...] = max_iou

    g_iota = jax.lax.broadcasted_iota(jnp.int32, (_G, _N), 0)
    onehot = (g_iota == final_arg).astype(jnp.float32)
    assigned = jax.lax.dot_general(
        gt_t_ref[...], onehot,
        dimension_numbers=(((1,), (0,)), ((), ())),
        precision=jax.lax.Precision.HIGHEST,
        preferred_element_type=jnp.float32,
    )

    src_tl = boxes_t_ref[0:2, :]
    wh = boxes_t_ref[2:4, :] - src_tl
    ctr = src_tl + 0.5 * wh
    dst_tl = assigned[0:2, :]
    base_wh = assigned[2:4, :] - dst_tl
    base_ctr = dst_tl + 0.5 * base_wh
    eps = jnp.float32(np.finfo(np.float32).eps)
    wh_c = jnp.maximum(wh, eps)
    dxy = (base_ctr - ctr) / wh_c
    dwh = jnp.log(base_wh / wh_c)
    loc = jnp.concatenate([dxy, dwh], axis=0)
    loc_ref[...] = loc * scores_ref[...]


def kernel(boxes, scores, gt_boxes):
    boxes_t = boxes.T
    scores2 = scores.reshape(1, _N)
    loc_t, label, max_ious = pl.pallas_call(
        _frnod_kernel,
        out_shape=[
            jax.ShapeDtypeStruct((4, _N), jnp.float32),
            jax.ShapeDtypeStruct((1, _N), jnp.int32),
            jax.ShapeDtypeStruct((1, _N), jnp.float32),
        ],
    )(boxes_t, scores2, gt_boxes, gt_boxes.T)
    return loc_t.T, label.reshape(_N), max_ious.reshape(_N)
